# baseline (speedup 1.0000x reference)
import functools

import jax
import jax.numpy as jnp
from jax import lax
from jax.experimental import pallas as pl
from jax.experimental.pallas import tpu as pltpu
from jax.experimental.pallas import tpu_sc as plsc

_L = 16
_NBIN = 1024
_CHUNK = 16


def _sc_quantize(x2d, lo_t, hi_t, midu_t, prm):
    nrows, d = x2d.shape
    nworkers = 32
    rows_per_w = nrows // nworkers
    nchunks = rows_per_w // _CHUNK
    nvec = d // _L

    mesh = plsc.VectorSubcoreMesh(core_axis_name="c", subcore_axis_name="s")

    @functools.partial(
        pl.kernel,
        mesh=mesh,
        out_type=jax.ShapeDtypeStruct((nrows, d), jnp.float32),
        compiler_params=pltpu.CompilerParams(needs_layout_passes=False),
        scratch_types=[
            pltpu.VMEM((_CHUNK, d), jnp.float32),
            pltpu.VMEM((_CHUNK, d), jnp.float32),
            pltpu.VMEM((_NBIN,), jnp.float32),
            pltpu.VMEM((_NBIN,), jnp.float32),
            pltpu.VMEM((_NBIN,), jnp.float32),
            pltpu.VMEM((_L,), jnp.float32),
        ],
    )
    def k(x_hbm, lo_hbm, hi_hbm, midu_hbm, prm_hbm, out_hbm,
          xbuf, obuf, lo_v, hi_v, midu_v, prm_v):
        wid = lax.axis_index("s") * 2 + lax.axis_index("c")
        pltpu.sync_copy(lo_hbm, lo_v)
        pltpu.sync_copy(hi_hbm, hi_v)
        pltpu.sync_copy(midu_hbm, midu_v)
        pltpu.sync_copy(prm_hbm, prm_v)

        pvec = prm_v[...]
        invh_v = jnp.full((_L,), pvec[0], jnp.float32)
        cu_v = jnp.full((_L,), pvec[1], jnp.float32)

        def row_body(r):
            def sq_body(i, acc):
                v = xbuf[r, pl.ds(i * _L, _L)]
                return acc + v * v

            acc = plsc.parallel_loop(
                0, nvec, 1, unroll=8,
                carry=jnp.zeros((_L,), jnp.float32))(sq_body)
            mean = jnp.sum(acc) * (1.0 / d)
            mv = jnp.full((_L,), mean, jnp.float32)
            bits = plsc.bitcast(mv, jnp.int32)
            y = plsc.bitcast((bits >> 1) + 0x1FBD1DF6, jnp.float32)
            y = 0.5 * (y + mv / y)
            y = 0.5 * (y + mv / y)
            y = 0.5 * (y + mv / y)
            stdv = y + 1e-8
            su = (1.0 / stdv) * invh_v

            @plsc.parallel_loop(0, nvec, 1, unroll=8)
            def q_body(i):
                v = xbuf[r, pl.ds(i * _L, _L)]
                u = v * su + cu_v
                ji = u.astype(jnp.int32)
                jc = jnp.minimum(jnp.maximum(ji, 0), _NBIN - 1)
                gm = plsc.load_gather(midu_v, [jc])
                glo = plsc.load_gather(lo_v, [jc])
                ghi = plsc.load_gather(hi_v, [jc])
                q = jnp.where(u > gm, ghi, glo)
                obuf[r, pl.ds(i * _L, _L)] = q * stdv

        def chunk_body(c, _):
            base = wid * rows_per_w + c * _CHUNK
            pltpu.sync_copy(x_hbm.at[pl.ds(base, _CHUNK)], xbuf)
            plsc.parallel_loop(0, _CHUNK, 1)(row_body)
            pltpu.sync_copy(obuf, out_hbm.at[pl.ds(base, _CHUNK)])
            return _

        lax.fori_loop(0, nchunks, chunk_body, 0)

    return k(x2d, lo_t, hi_t, midu_t, prm)


def _build_tables(levels):
    lv = levels.astype(jnp.float32)
    mids = 0.5 * (lv[1:] + lv[:-1])
    margin = 0.1
    a = mids[0] - margin
    width = (mids[-1] + margin) - a
    h = width / _NBIN
    invh = _NBIN / width
    edges = a + jnp.arange(_NBIN + 1, dtype=jnp.float32) * h
    loidx = jnp.searchsorted(mids, edges[:-1], side="left")
    hiidx = jnp.searchsorted(mids, edges[1:], side="left")
    lo_t = lv[loidx]
    hi_t = lv[hiidx]
    midu_t = jnp.where(hiidx > loidx,
                       (mids[jnp.clip(loidx, 0, 14)] - a) * invh,
                       jnp.inf).astype(jnp.float32)
    prm = jnp.zeros((_L,), jnp.float32)
    prm = prm.at[0].set(invh).at[1].set(-a * invh)
    return lo_t, hi_t, midu_t, prm


def kernel(x, levels):
    lo_t, hi_t, midu_t, prm = _build_tables(levels)
    b, s, d = x.shape
    out = _sc_quantize(x.reshape(b * s, d), lo_t, hi_t, midu_t, prm)
    return out.reshape(b, s, d)


# speedup vs baseline: 1.7350x; 1.7350x over previous
import functools

import jax
import jax.numpy as jnp
from jax import lax
from jax.experimental import pallas as pl
from jax.experimental.pallas import tpu as pltpu
from jax.experimental.pallas import tpu_sc as plsc

_L = 16
_NBIN = 1024
_CHUNK = 16


def _sc_quantize(x2d, lo_t, hi_t, midu_t, prm):
    nrows, d = x2d.shape
    nworkers = 32
    rows_per_w = nrows // nworkers
    nchunks = rows_per_w // _CHUNK
    nvec = d // _L

    mesh = plsc.VectorSubcoreMesh(core_axis_name="c", subcore_axis_name="s")

    @functools.partial(
        pl.kernel,
        mesh=mesh,
        out_type=jax.ShapeDtypeStruct((nrows, d), jnp.float32),
        compiler_params=pltpu.CompilerParams(needs_layout_passes=False),
        scratch_types=[
            pltpu.VMEM((_CHUNK, d), jnp.float32),
            pltpu.VMEM((_CHUNK, d), jnp.float32),
            pltpu.VMEM((_NBIN,), jnp.float32),
            pltpu.VMEM((_NBIN,), jnp.float32),
            pltpu.VMEM((_NBIN,), jnp.float32),
            pltpu.VMEM((_L,), jnp.float32),
        ],
    )
    def k(x_hbm, lo_hbm, hi_hbm, midu_hbm, prm_hbm, out_hbm,
          xbuf, obuf, lo_v, hi_v, midu_v, prm_v):
        wid = lax.axis_index("s") * 2 + lax.axis_index("c")
        pltpu.sync_copy(lo_hbm, lo_v)
        pltpu.sync_copy(hi_hbm, hi_v)
        pltpu.sync_copy(midu_hbm, midu_v)
        pltpu.sync_copy(prm_hbm, prm_v)

        pvec = prm_v[...]
        invh_v = jnp.full((_L,), pvec[0], jnp.float32)
        cu_v = jnp.full((_L,), pvec[1], jnp.float32)

        def row_body(r):
            def sq_body(i, acc):
                v = xbuf[r, pl.ds(i * _L, _L)]
                return acc + v * v

            acc = plsc.parallel_loop(
                0, nvec, 1, unroll=8,
                carry=jnp.zeros((_L,), jnp.float32))(sq_body)
            mean = jnp.sum(acc) * (1.0 / d)
            mv = jnp.full((_L,), mean, jnp.float32)
            bits = plsc.bitcast(mv, jnp.int32)
            y = plsc.bitcast((bits >> 1) + 0x1FBD1DF6, jnp.float32)
            y = 0.5 * (y + mv / y)
            y = 0.5 * (y + mv / y)
            y = 0.5 * (y + mv / y)
            stdv = y + 1e-8
            su = (1.0 / stdv) * invh_v

            @plsc.parallel_loop(0, nvec, 1, unroll=8)
            def q_body(i):
                v = xbuf[r, pl.ds(i * _L, _L)]
                u = v * su + cu_v
                ji = u.astype(jnp.int32)
                jc = jnp.minimum(jnp.maximum(ji, 0), _NBIN - 1)
                gm = plsc.load_gather(midu_v, [jc])
                glo = plsc.load_gather(lo_v, [jc])
                ghi = plsc.load_gather(hi_v, [jc])
                q = jnp.where(u > gm, ghi, glo)
                obuf[r, pl.ds(i * _L, _L)] = q * stdv

        def chunk_body(c, _):
            base = wid * rows_per_w + c * _CHUNK
            pltpu.sync_copy(x_hbm.at[pl.ds(base, _CHUNK)], xbuf)
            plsc.parallel_loop(0, _CHUNK, 1)(row_body)
            pltpu.sync_copy(obuf, out_hbm.at[pl.ds(base, _CHUNK)])
            return _

        lax.fori_loop(0, nchunks, chunk_body, 0)

    return k(x2d, lo_t, hi_t, midu_t, prm)


def _build_tables(levels):
    lv = levels.astype(jnp.float32)
    mids = 0.5 * (lv[1:] + lv[:-1])
    margin = 0.1
    a = mids[0] - margin
    width = (mids[-1] + margin) - a
    h = width / _NBIN
    invh = _NBIN / width
    edges = a + jnp.arange(_NBIN + 1, dtype=jnp.float32) * h
    loidx = jnp.sum(mids[None, :] < edges[:-1, None], axis=1)
    hiidx = jnp.sum(mids[None, :] < edges[1:, None], axis=1)
    lvl_iota = jnp.arange(16, dtype=jnp.int32)[None, :]
    lo_t = jnp.sum(lv[None, :] * (lvl_iota == loidx[:, None]), axis=1)
    hi_t = jnp.sum(lv[None, :] * (lvl_iota == hiidx[:, None]), axis=1)
    mid_iota = jnp.arange(15, dtype=jnp.int32)[None, :]
    mid_at_lo = jnp.sum(mids[None, :] *
                        (mid_iota == jnp.clip(loidx, 0, 14)[:, None]), axis=1)
    midu_t = jnp.where(hiidx > loidx,
                       (mid_at_lo - a) * invh,
                       jnp.inf).astype(jnp.float32)
    prm = jnp.zeros((_L,), jnp.float32)
    prm = prm.at[0].set(invh).at[1].set(-a * invh)
    return lo_t, hi_t, midu_t, prm


def kernel(x, levels):
    lo_t, hi_t, midu_t, prm = _build_tables(levels)
    b, s, d = x.shape
    out = _sc_quantize(x.reshape(b * s, d), lo_t, hi_t, midu_t, prm)
    return out.reshape(b, s, d)


# speedup vs baseline: 2.1911x; 1.2629x over previous
import functools

import jax
import jax.numpy as jnp
from jax import lax
from jax.experimental import pallas as pl
from jax.experimental.pallas import tpu as pltpu
from jax.experimental.pallas import tpu_sc as plsc

_L = 16
_NBIN = 1024
_CHUNK = 16


def _sc_quantize(x2d, lo_t, hi_t, midu_t, prm):
    nrows, d = x2d.shape
    nworkers = 32
    rows_per_w = nrows // nworkers
    nchunks = rows_per_w // _CHUNK
    nvec = d // _L

    mesh = plsc.VectorSubcoreMesh(core_axis_name="c", subcore_axis_name="s")

    @functools.partial(
        pl.kernel,
        mesh=mesh,
        out_type=jax.ShapeDtypeStruct((nrows, d), jnp.float32),
        compiler_params=pltpu.CompilerParams(needs_layout_passes=False),
        scratch_types=[
            pltpu.VMEM((_CHUNK, d), jnp.float32),
            pltpu.VMEM((_CHUNK, d), jnp.float32),
            pltpu.VMEM((_CHUNK, d), jnp.float32),
            pltpu.VMEM((_CHUNK, d), jnp.float32),
            pltpu.VMEM((_NBIN,), jnp.float32),
            pltpu.VMEM((_NBIN,), jnp.float32),
            pltpu.VMEM((_NBIN,), jnp.float32),
            pltpu.VMEM((_L,), jnp.float32),
            pltpu.SemaphoreType.DMA,
            pltpu.SemaphoreType.DMA,
            pltpu.SemaphoreType.DMA,
            pltpu.SemaphoreType.DMA,
        ],
    )
    def k(x_hbm, lo_hbm, hi_hbm, midu_hbm, prm_hbm, out_hbm,
          xb0, xb1, ob0, ob1, lo_v, hi_v, midu_v, prm_v,
          si0, si1, so0, so1):
        wid = lax.axis_index("s") * 2 + lax.axis_index("c")
        pltpu.sync_copy(lo_hbm, lo_v)
        pltpu.sync_copy(hi_hbm, hi_v)
        pltpu.sync_copy(midu_hbm, midu_v)
        pltpu.sync_copy(prm_hbm, prm_v)

        pvec = prm_v[...]
        invh_v = jnp.full((_L,), pvec[0], jnp.float32)
        cu_v = jnp.full((_L,), pvec[1], jnp.float32)

        def row_body_for(xbuf, obuf):
            return lambda r: _row(xbuf, obuf, r)

        def _row(xbuf, obuf, r):
            def sq_body(i, acc):
                v = xbuf[r, pl.ds(i * _L, _L)]
                return acc + v * v

            acc = plsc.parallel_loop(
                0, nvec, 1, unroll=8,
                carry=jnp.zeros((_L,), jnp.float32))(sq_body)
            mean = jnp.sum(acc) * (1.0 / d)
            mv = jnp.full((_L,), mean, jnp.float32)
            bits = plsc.bitcast(mv, jnp.int32)
            y = plsc.bitcast((bits >> 1) + 0x1FBD1DF6, jnp.float32)
            y = 0.5 * (y + mv / y)
            y = 0.5 * (y + mv / y)
            y = 0.5 * (y + mv / y)
            stdv = y + 1e-8
            su = (1.0 / stdv) * invh_v

            @plsc.parallel_loop(0, nvec, 1, unroll=8)
            def q_body(i):
                v = xbuf[r, pl.ds(i * _L, _L)]
                u = v * su + cu_v
                ji = u.astype(jnp.int32)
                jc = jnp.minimum(jnp.maximum(ji, 0), _NBIN - 1)
                gm = plsc.load_gather(midu_v, [jc])
                glo = plsc.load_gather(lo_v, [jc])
                ghi = plsc.load_gather(hi_v, [jc])
                q = jnp.where(u > gm, ghi, glo)
                obuf[r, pl.ds(i * _L, _L)] = q * stdv

        def in_slice(c):
            return x_hbm.at[pl.ds(wid * rows_per_w + c * _CHUNK, _CHUNK)]

        def out_slice(c):
            return out_hbm.at[pl.ds(wid * rows_per_w + c * _CHUNK, _CHUNK)]

        pltpu.make_async_copy(in_slice(0), xb0, si0).start()

        def pair_body(p, _):
            c0 = 2 * p
            pltpu.make_async_copy(in_slice(c0 + 1), xb1, si1).start()
            pltpu.make_async_copy(in_slice(c0), xb0, si0).wait()

            @pl.when(p > 0)
            def _wait_ob0():
                pltpu.make_async_copy(ob0, out_slice(c0), so0).wait()

            plsc.parallel_loop(0, _CHUNK, 1)(row_body_for(xb0, ob0))
            pltpu.make_async_copy(ob0, out_slice(c0), so0).start()

            @pl.when(c0 + 2 < nchunks)
            def _pf_xb0():
                pltpu.make_async_copy(in_slice(c0 + 2), xb0, si0).start()

            pltpu.make_async_copy(in_slice(c0 + 1), xb1, si1).wait()

            @pl.when(p > 0)
            def _wait_ob1():
                pltpu.make_async_copy(ob1, out_slice(c0 + 1), so1).wait()

            plsc.parallel_loop(0, _CHUNK, 1)(row_body_for(xb1, ob1))
            pltpu.make_async_copy(ob1, out_slice(c0 + 1), so1).start()
            return _

        lax.fori_loop(0, nchunks // 2, pair_body, 0)
        pltpu.make_async_copy(ob0, out_slice(nchunks - 2), so0).wait()
        pltpu.make_async_copy(ob1, out_slice(nchunks - 1), so1).wait()

    return k(x2d, lo_t, hi_t, midu_t, prm)


def _build_tables(levels):
    lv = levels.astype(jnp.float32)
    mids = 0.5 * (lv[1:] + lv[:-1])
    margin = 0.1
    a = mids[0] - margin
    width = (mids[-1] + margin) - a
    h = width / _NBIN
    invh = _NBIN / width
    edges = a + jnp.arange(_NBIN + 1, dtype=jnp.float32) * h
    loidx = jnp.sum(mids[None, :] < edges[:-1, None], axis=1)
    hiidx = jnp.sum(mids[None, :] < edges[1:, None], axis=1)
    lvl_iota = jnp.arange(16, dtype=jnp.int32)[None, :]
    lo_t = jnp.sum(lv[None, :] * (lvl_iota == loidx[:, None]), axis=1)
    hi_t = jnp.sum(lv[None, :] * (lvl_iota == hiidx[:, None]), axis=1)
    mid_iota = jnp.arange(15, dtype=jnp.int32)[None, :]
    mid_at_lo = jnp.sum(mids[None, :] *
                        (mid_iota == jnp.clip(loidx, 0, 14)[:, None]), axis=1)
    midu_t = jnp.where(hiidx > loidx,
                       (mid_at_lo - a) * invh,
                       jnp.inf).astype(jnp.float32)
    prm = jnp.zeros((_L,), jnp.float32)
    prm = prm.at[0].set(invh).at[1].set(-a * invh)
    return lo_t, hi_t, midu_t, prm


def kernel(x, levels):
    lo_t, hi_t, midu_t, prm = _build_tables(levels)
    b, s, d = x.shape
    out = _sc_quantize(x.reshape(b * s, d), lo_t, hi_t, midu_t, prm)
    return out.reshape(b, s, d)
